# Initial kernel scaffold; baseline (speedup 1.0000x reference)
import jax
import jax.numpy as jnp
from jax.experimental import pallas as pl


def kernel(x, table):
    raise NotImplementedError("write your pallas kernel here")


# speedup vs baseline: 1.1917x; 1.1917x over previous
import functools

import jax
import jax.numpy as jnp
from jax import lax
from jax.experimental import pallas as pl
from jax.experimental.pallas import tpu as pltpu
from jax.experimental.pallas import tpu_sc as plsc

CHUNK = 128
NBUF = 2


@functools.lru_cache(maxsize=None)
def _build(total, dim):
    info = plsc.get_sparse_core_info()
    nw = info.num_cores * info.num_subcores
    nc = info.num_cores

    n_chunks = total // CHUNK
    chunks_per_w = n_chunks // nw
    rows_per_w = chunks_per_w * CHUNK
    n_outer = chunks_per_w // NBUF

    mesh = plsc.VectorSubcoreMesh(core_axis_name="c", subcore_axis_name="s")

    @functools.partial(
        pl.kernel,
        mesh=mesh,
        compiler_params=pltpu.CompilerParams(use_tc_tiling_on_sc=False),
        out_type=jax.ShapeDtypeStruct((total, dim), jnp.float32),
        scratch_types=[
            pltpu.VMEM((chunks_per_w, CHUNK), jnp.int32),
            pltpu.VMEM((NBUF, CHUNK, dim), jnp.float32),
            pltpu.SemaphoreType.DMA,
            pltpu.SemaphoreType.DMA,
        ],
    )
    def gather_kernel(x_hbm, table_hbm, out_hbm, idx_v, rows_v, sem0, sem1):
        sems = (sem0, sem1)
        wid = lax.axis_index("s") * nc + lax.axis_index("c")
        base_row = wid * rows_per_w

        pltpu.sync_copy(x_hbm.at[wid], idx_v)

        for b in range(NBUF):
            pltpu.async_copy(table_hbm.at[idx_v.at[b]], rows_v.at[b], sems[b])

        def outer(g, carry):
            for b in range(NBUF):
                c = g * NBUF + b
                pltpu.make_async_copy(
                    table_hbm.at[idx_v.at[c]], rows_v.at[b], sems[b]
                ).wait()
                pltpu.sync_copy(
                    rows_v.at[b], out_hbm.at[pl.ds(base_row + c * CHUNK, CHUNK)]
                )
                pltpu.async_copy(
                    table_hbm.at[idx_v.at[c + NBUF]], rows_v.at[b], sems[b]
                )
            return carry

        lax.fori_loop(0, n_outer - 1, outer, 0)

        for b in range(NBUF):
            c = (n_outer - 1) * NBUF + b
            pltpu.make_async_copy(
                table_hbm.at[idx_v.at[c]], rows_v.at[b], sems[b]
            ).wait()
            pltpu.sync_copy(
                rows_v.at[b], out_hbm.at[pl.ds(base_row + c * CHUNK, CHUNK)]
            )

    return gather_kernel


def kernel(x, table):
    batch, fields = x.shape
    total = batch * fields
    dim = table.shape[1]
    nw = 32
    xf = x.reshape(nw, total // (nw * CHUNK), CHUNK)
    out = _build(total, dim)(xf, table)
    return out.reshape(batch, fields, dim)


# speedup vs baseline: 1.2184x; 1.0224x over previous
import functools

import jax
import jax.numpy as jnp
from jax import lax
from jax.experimental import pallas as pl
from jax.experimental.pallas import tpu as pltpu
from jax.experimental.pallas import tpu_sc as plsc

CHUNK = 128
NBUF = 4


@functools.lru_cache(maxsize=None)
def _build(total, dim):
    info = plsc.get_sparse_core_info()
    nw = info.num_cores * info.num_subcores
    nc = info.num_cores

    n_chunks = total // CHUNK
    chunks_per_w = n_chunks // nw
    rows_per_w = chunks_per_w * CHUNK
    n_outer = chunks_per_w // NBUF
    rem = chunks_per_w - n_outer * NBUF

    mesh = plsc.VectorSubcoreMesh(core_axis_name="c", subcore_axis_name="s")

    @functools.partial(
        pl.kernel,
        mesh=mesh,
        compiler_params=pltpu.CompilerParams(use_tc_tiling_on_sc=False),
        out_type=jax.ShapeDtypeStruct((total, dim), jnp.float32),
        scratch_types=[
            pltpu.VMEM((chunks_per_w, CHUNK), jnp.int32),
            pltpu.VMEM((NBUF, CHUNK, dim), jnp.float32),
        ]
        + [pltpu.SemaphoreType.DMA] * (2 * NBUF),
    )
    def gather_kernel(x_hbm, table_hbm, out_hbm, idx_v, rows_v, *sems):
        gsems, osems = sems[:NBUF], sems[NBUF:]
        wid = lax.axis_index("s") * nc + lax.axis_index("c")
        base_row = wid * rows_per_w

        def out_slot(c):
            return out_hbm.at[pl.ds(base_row + c * CHUNK, CHUNK)]

        pltpu.sync_copy(x_hbm.at[wid], idx_v)

        for b in range(NBUF):
            pltpu.async_copy(table_hbm.at[idx_v.at[b]], rows_v.at[b], gsems[b])

        def outer(g, carry):
            for b in range(NBUF):
                c = g * NBUF + b
                pltpu.make_async_copy(
                    table_hbm.at[idx_v.at[c]], rows_v.at[b], gsems[b]
                ).wait()
                pltpu.async_copy(rows_v.at[b], out_slot(c), osems[b])
                nxt = c + NBUF

                @pl.when(nxt < chunks_per_w)
                def _():
                    pltpu.make_async_copy(rows_v.at[b], out_slot(c), osems[b]).wait()
                    pltpu.async_copy(
                        table_hbm.at[idx_v.at[nxt]], rows_v.at[b], gsems[b]
                    )

            return carry

        lax.fori_loop(0, n_outer, outer, 0)

        for b in range(rem):
            c = n_outer * NBUF + b
            pltpu.make_async_copy(
                table_hbm.at[idx_v.at[c]], rows_v.at[b], gsems[b]
            ).wait()
            pltpu.async_copy(rows_v.at[b], out_slot(c), osems[b])

        for b in range(NBUF):
            c = chunks_per_w - NBUF + b
            pltpu.make_async_copy(rows_v.at[b], out_slot(c), osems[b]).wait()

    return gather_kernel


def kernel(x, table):
    batch, fields = x.shape
    total = batch * fields
    dim = table.shape[1]
    nw = 32
    xf = x.reshape(nw, total // (nw * CHUNK), CHUNK)
    out = _build(total, dim)(xf, table)
    return out.reshape(batch, fields, dim)
